# baseline (speedup 1.0000x reference)
import functools

import jax
import jax.numpy as jnp
from jax import lax
from jax.experimental import pallas as pl
from jax.experimental.pallas import tpu as pltpu
from jax.experimental.pallas import tpu_sc as plsc

HIDDEN = 1024
SEQ = 2048
BATCH = 4
TOK = BATCH * SEQ
EPS = 1e-12

L = 16
NVEC = HIDDEN // L

NC = 2
NS = 16
NW = NC * NS
TPW = TOK // NW
C = 32
NCHUNK = TPW // C


def _rsqrt_nr(v16):
    bits = lax.bitcast_convert_type(v16, jnp.int32)
    y = lax.bitcast_convert_type(
        jnp.int32(0x5F3759DF) - lax.shift_right_arithmetic(bits, 1),
        jnp.float32)
    for _ in range(3):
        y = y * (1.5 - 0.5 * v16 * y * y)
    return y


@functools.partial(
    pl.kernel,
    mesh=plsc.VectorSubcoreMesh(core_axis_name="c", subcore_axis_name="s"),
    compiler_params=pltpu.CompilerParams(needs_layout_passes=False),
    out_type=jax.ShapeDtypeStruct((TOK, HIDDEN), jnp.float32),
    scratch_types=[
        pltpu.VMEM((C, HIDDEN), jnp.float32),
        pltpu.VMEM((C, HIDDEN), jnp.float32),
        pltpu.VMEM((2, HIDDEN), jnp.float32),
        pltpu.VMEM((TPW,), jnp.int32),
        pltpu.VMEM((TPW,), jnp.int32),
        pltpu.VMEM((HIDDEN,), jnp.float32),
        pltpu.VMEM((HIDDEN,), jnp.float32),
        pltpu.SemaphoreType.DMA,
    ],
)
def _embed_ln(ids_hbm, tids_hbm, word_hbm, pos_hbm, type_hbm, gamma_hbm,
              beta_hbm, out_hbm, buf, posbuf, typebuf, wid_v, tid_v,
              gamma_v, beta_v, sem):
    wid = lax.axis_index("s") * NC + lax.axis_index("c")
    tok_base = wid * TPW
    pos_base = lax.rem(tok_base, SEQ)

    pltpu.sync_copy(ids_hbm.at[pl.ds(tok_base, TPW)], wid_v)
    pltpu.sync_copy(tids_hbm.at[pl.ds(tok_base, TPW)], tid_v)
    pltpu.sync_copy(type_hbm, typebuf)
    pltpu.sync_copy(gamma_hbm, gamma_v)
    pltpu.sync_copy(beta_hbm, beta_v)

    iota = lax.iota(jnp.int32, L)
    zero = jnp.zeros((L,), jnp.float32)

    for ci in range(NCHUNK):
        pltpu.async_copy(word_hbm.at[wid_v.at[pl.ds(ci * C, C)]], buf,
                         sem).wait()
        pltpu.sync_copy(pos_hbm.at[pl.ds(pos_base + ci * C, C)], posbuf)

        def token_body(t, _):
            grp = ci * C + (t // L) * L
            tid16 = tid_v[pl.ds(grp, L)].astype(jnp.float32)
            tid_t = jnp.sum(
                jnp.where(iota == lax.rem(t, L), tid16, 0.0)).astype(jnp.int32)

            def p1(j, carry):
                s1, s2 = carry
                x = (buf[t, pl.ds(j * L, L)]
                     + posbuf[t, pl.ds(j * L, L)]
                     + typebuf[tid_t, pl.ds(j * L, L)])
                buf[t, pl.ds(j * L, L)] = x
                return s1 + x, s2 + x * x

            s1, s2 = lax.fori_loop(0, NVEC, p1, (zero, zero))
            tot = jnp.sum(s1)
            tot2 = jnp.sum(s2)
            mean = tot * (1.0 / HIDDEN)
            var = tot2 * (1.0 / HIDDEN) - mean * mean
            inv = _rsqrt_nr(lax.broadcast(var + EPS, (L,)))
            mean_v = lax.broadcast(mean, (L,))

            def p2(j, _):
                x = buf[t, pl.ds(j * L, L)]
                g = gamma_v[pl.ds(j * L, L)]
                b = beta_v[pl.ds(j * L, L)]
                buf[t, pl.ds(j * L, L)] = (x - mean_v) * inv * g + b
                return 0

            lax.fori_loop(0, NVEC, p2, 0)
            return 0

        lax.fori_loop(0, C, token_body, 0)
        pltpu.sync_copy(buf, out_hbm.at[pl.ds(tok_base + ci * C, C)])


def kernel(input_ids, token_type_ids, word_emb, pos_emb, type_emb, gamma,
           beta):
    ids = input_ids.reshape(-1).astype(jnp.int32)
    tids = token_type_ids.reshape(-1).astype(jnp.int32)
    out = _embed_ln(ids, tids, word_emb, pos_emb, type_emb, gamma, beta)
    return out.reshape(input_ids.shape[0], input_ids.shape[1], HIDDEN)
